# Initial kernel scaffold; baseline (speedup 1.0000x reference)
import jax
import jax.numpy as jnp
from jax.experimental import pallas as pl


def kernel(features, num_voxels, coors, params):
    raise NotImplementedError("write your pallas kernel here")


# speedup vs baseline: 1.3726x; 1.3726x over previous
import jax
import jax.numpy as jnp
from jax.experimental import pallas as pl

V = 32
NPOINT = 10
VX = 0.2
VY = 0.2
X_OFFSET = VX / 2 + 0.0
Y_OFFSET = VY / 2 + (-40.0)
RADIUS_LIST = [0.32, 0.36, 0.4]
NSAMPLE_LIST = [6, 8, 10]
BN_EPS = 1e-5

PB = 8


def _fold(p):
    s = p["gamma"] / jnp.sqrt(1.0 + BN_EPS)
    w = (p["w"] * s[:, None]).T
    b = p["b"] * s + p["beta"]
    return w, b[None, :]


def _sa0_kernel_body(feat_ref, nv_ref, coors_ref,
                     w1_ref, b1_ref, w2_ref, b2_ref, w3_ref, b3_ref,
                     w4a_ref, b4a_ref, w4b_ref, b4b_ref, w4c_ref, b4c_ref,
                     s1w_ref, s1b_ref, s2w_ref, s2b_ref, s3w_ref, s3b_ref,
                     out_ref):
    f32 = jnp.float32
    f = feat_ref[...]
    nv = nv_ref[...].astype(f32)
    coors = coors_ref[...].astype(f32)

    pmean = jnp.sum(f[:, :, :3], axis=1, keepdims=True) / nv[:, :, None]
    fcl = f[:, :, :3] - pmean
    fc0 = f[:, :, 0] - (coors[:, 3][:, None] * VX + X_OFFSET)
    fc1 = f[:, :, 1] - (coors[:, 2][:, None] * VY + Y_OFFSET)
    feats9 = jnp.concatenate(
        [f, fcl, fc0[:, :, None], fc1[:, :, None]], axis=-1)
    jidx = jax.lax.broadcasted_iota(jnp.int32, (PB, V), 1)
    vmask = (nv > jidx.astype(f32)).astype(f32)
    feats9 = feats9 * vmask[:, :, None]

    xyz = feats9[:, :, 0:3]
    data6 = feats9[:, :, 3:9]

    distance = jnp.full((PB, V), 1e10, dtype=f32)
    farth = jnp.zeros((PB, 1), dtype=jnp.int32)
    cents = []
    for _ in range(NPOINT):
        oh = (jidx == farth).astype(f32)
        centroid = jnp.sum(xyz * oh[:, :, None], axis=1)
        cents.append(centroid)
        dx = xyz[:, :, 0] - centroid[:, 0][:, None]
        dy = xyz[:, :, 1] - centroid[:, 1][:, None]
        dz = xyz[:, :, 2] - centroid[:, 2][:, None]
        d = (dx * dx + dy * dy) + dz * dz
        distance = jnp.minimum(distance, d)
        m = jnp.max(distance, axis=1, keepdims=True)
        farth = jnp.min(jnp.where(distance == m, jidx, V), axis=1,
                        keepdims=True)
    new_xyz = jnp.stack(cents, axis=1)

    rows = PB * NPOINT * V
    rel = xyz[:, None, :, :] - new_xyz[:, :, None, :]
    d6b = jnp.broadcast_to(data6[:, None, :, :], (PB, NPOINT, V, 6))
    x0 = jnp.concatenate([d6b, rel], axis=-1).reshape(rows, 9)

    rx, ry, rz = x0[:, 6:7], x0[:, 7:8], x0[:, 8:9]
    sqrd_rows = (rx * rx + ry * ry) + rz * rz
    s4 = ((rel[..., 0] * rel[..., 0] + rel[..., 1] * rel[..., 1])
          + rel[..., 2] * rel[..., 2])
    s2d = s4.reshape(PB * NPOINT, V)

    jrow = jax.lax.broadcasted_iota(jnp.int32, (V, V), 0)
    jcol = jax.lax.broadcasted_iota(jnp.int32, (V, V), 1)
    ltr = jnp.broadcast_to(((jcol < jrow).astype(f32))[None, :, :],
                           (PB * NPOINT, V, V)).reshape(rows, V)

    sels = []
    for r, ns in zip(RADIUS_LIST, NSAMPLE_LIST):
        inr2 = (s2d <= r * r).astype(f32)
        mrows = jnp.broadcast_to(inr2[:, None, :],
                                 (PB * NPOINT, V, V)).reshape(rows, V)
        rank = jnp.sum(mrows * ltr, axis=1, keepdims=True)
        inr_rows = (sqrd_rows <= r * r).astype(f32)
        sels.append(inr_rows * (rank < float(ns)).astype(f32))

    h = jnp.maximum(jnp.dot(x0, w1_ref[...], preferred_element_type=f32)
                    + b1_ref[...], 0.0)
    h = jnp.maximum(jnp.dot(h, w2_ref[...], preferred_element_type=f32)
                    + b2_ref[...], 0.0)
    h = jnp.maximum(jnp.dot(h, w3_ref[...], preferred_element_type=f32)
                    + b3_ref[...], 0.0)
    y0 = jnp.maximum(jnp.dot(h[:, 0:48], w4a_ref[...],
                             preferred_element_type=f32) + b4a_ref[...], 0.0)
    y1 = jnp.maximum(jnp.dot(h[:, 48:96], w4b_ref[...],
                             preferred_element_type=f32) + b4b_ref[...], 0.0)
    y2 = jnp.maximum(jnp.dot(h[:, 96:192], w4c_ref[...],
                             preferred_element_type=f32) + b4c_ref[...], 0.0)

    pooled = []
    for y, sel in zip((y0, y1, y2), sels):
        ym = (y * sel).reshape(PB * NPOINT, V, y.shape[-1])
        pooled.append(jnp.max(ym, axis=1))
    sa0_out = jnp.concatenate(pooled, axis=-1)

    xin = jnp.concatenate([new_xyz.reshape(PB * NPOINT, 3), sa0_out], axis=-1)
    g = jnp.maximum(jnp.dot(xin, s1w_ref[...], preferred_element_type=f32)
                    + s1b_ref[...], 0.0)
    g = jnp.maximum(jnp.dot(g, s2w_ref[...], preferred_element_type=f32)
                    + s2b_ref[...], 0.0)
    g = jnp.maximum(jnp.dot(g, s3w_ref[...], preferred_element_type=f32)
                    + s3b_ref[...], 0.0)
    out_ref[...] = jnp.max(g.reshape(PB, NPOINT, 128), axis=1)


def kernel(features, num_voxels, coors, params):
    p = features.shape[0]
    assert p % PB == 0
    grid = p // PB

    sa0 = [[_fold(layer) for layer in branch] for branch in params["sa0"]]
    sa1 = [_fold(layer) for layer in params["sa1"]]

    w1 = jnp.concatenate([sa0[0][0][0], sa0[1][0][0], sa0[2][0][0]], axis=1)
    b1 = jnp.concatenate([sa0[0][0][1], sa0[1][0][1], sa0[2][0][1]], axis=1)

    def blockdiag(mats):
        rows = sum(m.shape[0] for m in mats)
        cols = sum(m.shape[1] for m in mats)
        out = jnp.zeros((rows, cols), jnp.float32)
        r = c = 0
        for m in mats:
            out = out.at[r:r + m.shape[0], c:c + m.shape[1]].set(m)
            r += m.shape[0]
            c += m.shape[1]
        return out

    w2 = blockdiag([sa0[0][1][0], sa0[1][1][0], sa0[2][1][0]])
    b2 = jnp.concatenate([sa0[0][1][1], sa0[1][1][1], sa0[2][1][1]], axis=1)
    w3 = blockdiag([sa0[0][2][0], sa0[1][2][0], sa0[2][2][0]])
    b3 = jnp.concatenate([sa0[0][2][1], sa0[1][2][1], sa0[2][2][1]], axis=1)
    w4a, b4a = sa0[0][3]
    w4b, b4b = sa0[1][3]
    w4c, b4c = sa0[2][3]
    (s1w, s1b), (s2w, s2b), (s3w, s3b) = sa1

    nv2 = num_voxels.reshape(p, 1)

    def bcast(a):
        return pl.BlockSpec(a.shape, lambda i: (0,) * a.ndim)

    wlist = [w1, b1, w2, b2, w3, b3, w4a, b4a, w4b, b4b, w4c, b4c,
             s1w, s1b, s2w, s2b, s3w, s3b]

    out = pl.pallas_call(
        _sa0_kernel_body,
        grid=(grid,),
        in_specs=[
            pl.BlockSpec((PB, V, 4), lambda i: (i, 0, 0)),
            pl.BlockSpec((PB, 1), lambda i: (i, 0)),
            pl.BlockSpec((PB, 4), lambda i: (i, 0)),
        ] + [bcast(a) for a in wlist],
        out_specs=pl.BlockSpec((PB, 128), lambda i: (i, 0)),
        out_shape=jax.ShapeDtypeStruct((p, 128), jnp.float32),
    )(features, nv2, coors, *wlist)
    return out


# speedup vs baseline: 7.7851x; 5.6719x over previous
import jax
import jax.numpy as jnp
from jax.experimental import pallas as pl

V = 32
NPOINT = 10
VX = 0.2
VY = 0.2
X_OFFSET = VX / 2 + 0.0
Y_OFFSET = VY / 2 + (-40.0)
RADIUS_LIST = [0.32, 0.36, 0.4]
NSAMPLE_LIST = [6, 8, 10]
BN_EPS = 1e-5

PB = 8


def _fold(p):
    s = p["gamma"] / jnp.sqrt(1.0 + BN_EPS)
    w = (p["w"] * s[:, None]).T
    b = p["b"] * s + p["beta"]
    return w, b[None, :]


def _sa0_kernel_body(feat_ref, nv_ref, coors_ref,
                     w1_ref, w1c_ref, b1_ref, w2_ref, b2_ref, w3_ref, b3_ref,
                     w4a_ref, b4a_ref, w4b_ref, b4b_ref, w4c_ref, b4c_ref,
                     s1w_ref, s1b_ref, s2w_ref, s2b_ref, s3w_ref, s3b_ref,
                     out_ref):
    f32 = jnp.float32
    f = feat_ref[...]
    nv = nv_ref[...].astype(f32)
    coors = coors_ref[...].astype(f32)

    pmean = jnp.sum(f[:, :, :3], axis=1, keepdims=True) / nv[:, :, None]
    fcl = f[:, :, :3] - pmean
    fc0 = f[:, :, 0] - (coors[:, 3][:, None] * VX + X_OFFSET)
    fc1 = f[:, :, 1] - (coors[:, 2][:, None] * VY + Y_OFFSET)
    feats9 = jnp.concatenate(
        [f, fcl, fc0[:, :, None], fc1[:, :, None]], axis=-1)
    jidx = jax.lax.broadcasted_iota(jnp.int32, (PB, V), 1)
    vmask = (nv > jidx.astype(f32)).astype(f32)
    feats9 = feats9 * vmask[:, :, None]

    xyz = feats9[:, :, 0:3]
    data6 = feats9[:, :, 3:9]

    distance = jnp.full((PB, V), 1e10, dtype=f32)
    farth = jnp.zeros((PB, 1), dtype=jnp.int32)
    cents = []
    for _ in range(NPOINT):
        oh = (jidx == farth).astype(f32)
        centroid = jnp.sum(xyz * oh[:, :, None], axis=1)
        cents.append(centroid)
        dx = xyz[:, :, 0] - centroid[:, 0][:, None]
        dy = xyz[:, :, 1] - centroid[:, 1][:, None]
        dz = xyz[:, :, 2] - centroid[:, 2][:, None]
        d = (dx * dx + dy * dy) + dz * dz
        distance = jnp.minimum(distance, d)
        m = jnp.max(distance, axis=1, keepdims=True)
        farth = jnp.min(jnp.where(distance == m, jidx, V), axis=1,
                        keepdims=True)
    new_xyz = jnp.stack(cents, axis=1)

    dds = []
    for c in range(3):
        nxc = new_xyz[:, :, c]
        xc = xyz[:, :, c]
        dds.append(nxc[:, :, None] - xc[:, None, :])
    s4 = (dds[0] * dds[0] + dds[1] * dds[1]) + dds[2] * dds[2]
    s2d = s4.reshape(PB * NPOINT, V)

    jrow = jax.lax.broadcasted_iota(jnp.int32, (V, V), 0)
    jcol = jax.lax.broadcasted_iota(jnp.int32, (V, V), 1)
    lt = (jrow < jcol).astype(f32)

    sels = []
    for r, ns in zip(RADIUS_LIST, NSAMPLE_LIST):
        inr = (s2d <= r * r).astype(f32)
        rank = jnp.dot(inr, lt, preferred_element_type=f32)
        sels.append(inr * (rank < float(ns)).astype(f32))

    rows = PB * NPOINT * V
    pp = jnp.dot(feats9.reshape(PB * V, 9), w1_ref[...],
                 preferred_element_type=f32)
    pc = jnp.dot(new_xyz.reshape(PB * NPOINT, 3), w1c_ref[...],
                 preferred_element_type=f32)
    h = jnp.maximum(pp.reshape(PB, 1, V, 80)
                    - pc.reshape(PB, NPOINT, 1, 80)
                    + b1_ref[...], 0.0).reshape(rows, 80)

    h = jnp.maximum(jnp.dot(h, w2_ref[...], preferred_element_type=f32)
                    + b2_ref[...], 0.0)
    h = jnp.maximum(jnp.dot(h, w3_ref[...], preferred_element_type=f32)
                    + b3_ref[...], 0.0)

    pooled = []
    for (lo, hi), w4, b4, sel in zip(((0, 48), (48, 96), (96, 192)),
                                     (w4a_ref, w4b_ref, w4c_ref),
                                     (b4a_ref, b4b_ref, b4c_ref), sels):
        y = jnp.maximum(jnp.dot(h[:, lo:hi], w4[...],
                                preferred_element_type=f32) + b4[...], 0.0)
        ym = y.reshape(PB * NPOINT, V, y.shape[-1]) * sel[:, :, None]
        pooled.append(jnp.max(ym, axis=1))
    sa0_out = jnp.concatenate(pooled, axis=-1)

    xin = jnp.concatenate([new_xyz.reshape(PB * NPOINT, 3), sa0_out], axis=-1)
    g = jnp.maximum(jnp.dot(xin, s1w_ref[...], preferred_element_type=f32)
                    + s1b_ref[...], 0.0)
    g = jnp.maximum(jnp.dot(g, s2w_ref[...], preferred_element_type=f32)
                    + s2b_ref[...], 0.0)
    g = jnp.maximum(jnp.dot(g, s3w_ref[...], preferred_element_type=f32)
                    + s3b_ref[...], 0.0)
    out_ref[...] = jnp.max(g.reshape(PB, NPOINT, 128), axis=1)


def kernel(features, num_voxels, coors, params):
    p = features.shape[0]
    assert p % PB == 0
    grid = p // PB

    sa0 = [[_fold(layer) for layer in branch] for branch in params["sa0"]]
    sa1 = [_fold(layer) for layer in params["sa1"]]

    w1 = jnp.concatenate([sa0[0][0][0], sa0[1][0][0], sa0[2][0][0]], axis=1)
    b1 = jnp.concatenate([sa0[0][0][1], sa0[1][0][1], sa0[2][0][1]], axis=1)
    w1c = w1[6:9]
    w1 = jnp.concatenate([w1[6:9], w1[0:6]], axis=0)

    def blockdiag(mats):
        rows = sum(m.shape[0] for m in mats)
        cols = sum(m.shape[1] for m in mats)
        out = jnp.zeros((rows, cols), jnp.float32)
        r = c = 0
        for m in mats:
            out = out.at[r:r + m.shape[0], c:c + m.shape[1]].set(m)
            r += m.shape[0]
            c += m.shape[1]
        return out

    w2 = blockdiag([sa0[0][1][0], sa0[1][1][0], sa0[2][1][0]])
    b2 = jnp.concatenate([sa0[0][1][1], sa0[1][1][1], sa0[2][1][1]], axis=1)
    w3 = blockdiag([sa0[0][2][0], sa0[1][2][0], sa0[2][2][0]])
    b3 = jnp.concatenate([sa0[0][2][1], sa0[1][2][1], sa0[2][2][1]], axis=1)
    w4a, b4a = sa0[0][3]
    w4b, b4b = sa0[1][3]
    w4c, b4c = sa0[2][3]
    (s1w, s1b), (s2w, s2b), (s3w, s3b) = sa1

    nv2 = num_voxels.reshape(p, 1)

    def bcast(a):
        return pl.BlockSpec(a.shape, lambda i: (0,) * a.ndim)

    wlist = [w1, w1c, b1, w2, b2, w3, b3, w4a, b4a, w4b, b4b, w4c, b4c,
             s1w, s1b, s2w, s2b, s3w, s3b]

    out = pl.pallas_call(
        _sa0_kernel_body,
        grid=(grid,),
        in_specs=[
            pl.BlockSpec((PB, V, 4), lambda i: (i, 0, 0)),
            pl.BlockSpec((PB, 1), lambda i: (i, 0)),
            pl.BlockSpec((PB, 4), lambda i: (i, 0)),
        ] + [bcast(a) for a in wlist],
        out_specs=pl.BlockSpec((PB, 128), lambda i: (i, 0)),
        out_shape=jax.ShapeDtypeStruct((p, 128), jnp.float32),
    )(features, nv2, coors, *wlist)
    return out


# speedup vs baseline: 11.6830x; 1.5007x over previous
import jax
import jax.numpy as jnp
from jax.experimental import pallas as pl

V = 32
NPOINT = 10
VX = 0.2
VY = 0.2
X_OFFSET = VX / 2 + 0.0
Y_OFFSET = VY / 2 + (-40.0)
RADIUS_LIST = [0.32, 0.36, 0.4]
NSAMPLE_LIST = [6, 8, 10]
BN_EPS = 1e-5

PB = 16


def _fold(p):
    s = p["gamma"] / jnp.sqrt(1.0 + BN_EPS)
    w = (p["w"] * s[:, None]).T
    b = p["b"] * s + p["beta"]
    return w, b[None, :]


def _sa0_kernel_body(feat_ref, nv_ref, coors_ref,
                     w1_ref, w1c_ref, b1_ref, w2_ref, b2_ref, w3_ref, b3_ref,
                     w4a_ref, b4a_ref, w4b_ref, b4b_ref, w4c_ref, b4c_ref,
                     s1w_ref, s1b_ref, s2w_ref, s2b_ref, s3w_ref, s3b_ref,
                     out_ref):
    f32 = jnp.float32
    f = feat_ref[...]
    nv = nv_ref[...].astype(f32)
    coors = coors_ref[...].astype(f32)

    pmean = jnp.sum(f[:, :, :3], axis=1, keepdims=True) / nv[:, :, None]
    fcl = f[:, :, :3] - pmean
    fc0 = f[:, :, 0] - (coors[:, 3][:, None] * VX + X_OFFSET)
    fc1 = f[:, :, 1] - (coors[:, 2][:, None] * VY + Y_OFFSET)
    feats9 = jnp.concatenate(
        [f, fcl, fc0[:, :, None], fc1[:, :, None]], axis=-1)
    jidx = jax.lax.broadcasted_iota(jnp.int32, (PB, V), 1)
    vmask = (nv > jidx.astype(f32)).astype(f32)
    feats9 = feats9 * vmask[:, :, None]

    xyz = feats9[:, :, 0:3]
    data6 = feats9[:, :, 3:9]

    distance = jnp.full((PB, V), 1e10, dtype=f32)
    farth = jnp.zeros((PB, 1), dtype=jnp.int32)
    cents = []
    for _ in range(NPOINT):
        oh = (jidx == farth).astype(f32)
        centroid = jnp.sum(xyz * oh[:, :, None], axis=1)
        cents.append(centroid)
        dx = xyz[:, :, 0] - centroid[:, 0][:, None]
        dy = xyz[:, :, 1] - centroid[:, 1][:, None]
        dz = xyz[:, :, 2] - centroid[:, 2][:, None]
        d = (dx * dx + dy * dy) + dz * dz
        distance = jnp.minimum(distance, d)
        m = jnp.max(distance, axis=1, keepdims=True)
        farth = jnp.min(jnp.where(distance == m, jidx, V), axis=1,
                        keepdims=True)
    new_xyz = jnp.stack(cents, axis=1)

    dds = []
    for c in range(3):
        nxc = new_xyz[:, :, c]
        xc = xyz[:, :, c]
        dds.append(nxc[:, :, None] - xc[:, None, :])
    s4 = (dds[0] * dds[0] + dds[1] * dds[1]) + dds[2] * dds[2]
    s2d = s4.reshape(PB * NPOINT, V)

    jrow = jax.lax.broadcasted_iota(jnp.int32, (V, V), 0)
    jcol = jax.lax.broadcasted_iota(jnp.int32, (V, V), 1)
    lt = (jrow < jcol).astype(f32)

    sels = []
    for r, ns in zip(RADIUS_LIST, NSAMPLE_LIST):
        inr = (s2d <= r * r).astype(f32)
        rank = jnp.dot(inr, lt, preferred_element_type=f32)
        sels.append(inr * (rank < float(ns)).astype(f32))

    rows = PB * NPOINT * V
    pp = jnp.dot(feats9.reshape(PB * V, 9), w1_ref[...],
                 preferred_element_type=f32)
    pc = jnp.dot(new_xyz.reshape(PB * NPOINT, 3), w1c_ref[...],
                 preferred_element_type=f32)
    h = jnp.maximum(pp.reshape(PB, 1, V, 80)
                    - pc.reshape(PB, NPOINT, 1, 80)
                    + b1_ref[...], 0.0).reshape(rows, 80)

    h = jnp.maximum(jnp.dot(h, w2_ref[...], preferred_element_type=f32)
                    + b2_ref[...], 0.0)
    h = jnp.maximum(jnp.dot(h, w3_ref[...], preferred_element_type=f32)
                    + b3_ref[...], 0.0)

    pooled = []
    for (lo, hi), w4, b4, sel in zip(((0, 48), (48, 96), (96, 192)),
                                     (w4a_ref, w4b_ref, w4c_ref),
                                     (b4a_ref, b4b_ref, b4c_ref), sels):
        y = jnp.maximum(jnp.dot(h[:, lo:hi], w4[...],
                                preferred_element_type=f32) + b4[...], 0.0)
        ym = y.reshape(PB * NPOINT, V, y.shape[-1]) * sel[:, :, None]
        pooled.append(jnp.max(ym, axis=1))
    sa0_out = jnp.concatenate(pooled, axis=-1)

    xin = jnp.concatenate([new_xyz.reshape(PB * NPOINT, 3), sa0_out], axis=-1)
    g = jnp.maximum(jnp.dot(xin, s1w_ref[...], preferred_element_type=f32)
                    + s1b_ref[...], 0.0)
    g = jnp.maximum(jnp.dot(g, s2w_ref[...], preferred_element_type=f32)
                    + s2b_ref[...], 0.0)
    g = jnp.maximum(jnp.dot(g, s3w_ref[...], preferred_element_type=f32)
                    + s3b_ref[...], 0.0)
    out_ref[...] = jnp.max(g.reshape(PB, NPOINT, 128), axis=1)


def kernel(features, num_voxels, coors, params):
    p = features.shape[0]
    assert p % PB == 0
    grid = p // PB

    sa0 = [[_fold(layer) for layer in branch] for branch in params["sa0"]]
    sa1 = [_fold(layer) for layer in params["sa1"]]

    w1 = jnp.concatenate([sa0[0][0][0], sa0[1][0][0], sa0[2][0][0]], axis=1)
    b1 = jnp.concatenate([sa0[0][0][1], sa0[1][0][1], sa0[2][0][1]], axis=1)
    w1c = w1[6:9]
    w1 = jnp.concatenate([w1[6:9], w1[0:6]], axis=0)

    def blockdiag(mats):
        rows = sum(m.shape[0] for m in mats)
        cols = sum(m.shape[1] for m in mats)
        out = jnp.zeros((rows, cols), jnp.float32)
        r = c = 0
        for m in mats:
            out = out.at[r:r + m.shape[0], c:c + m.shape[1]].set(m)
            r += m.shape[0]
            c += m.shape[1]
        return out

    w2 = blockdiag([sa0[0][1][0], sa0[1][1][0], sa0[2][1][0]])
    b2 = jnp.concatenate([sa0[0][1][1], sa0[1][1][1], sa0[2][1][1]], axis=1)
    w3 = blockdiag([sa0[0][2][0], sa0[1][2][0], sa0[2][2][0]])
    b3 = jnp.concatenate([sa0[0][2][1], sa0[1][2][1], sa0[2][2][1]], axis=1)
    w4a, b4a = sa0[0][3]
    w4b, b4b = sa0[1][3]
    w4c, b4c = sa0[2][3]
    (s1w, s1b), (s2w, s2b), (s3w, s3b) = sa1

    nv2 = num_voxels.reshape(p, 1)

    def bcast(a):
        return pl.BlockSpec(a.shape, lambda i: (0,) * a.ndim)

    wlist = [w1, w1c, b1, w2, b2, w3, b3, w4a, b4a, w4b, b4b, w4c, b4c,
             s1w, s1b, s2w, s2b, s3w, s3b]

    out = pl.pallas_call(
        _sa0_kernel_body,
        grid=(grid,),
        in_specs=[
            pl.BlockSpec((PB, V, 4), lambda i: (i, 0, 0)),
            pl.BlockSpec((PB, 1), lambda i: (i, 0)),
            pl.BlockSpec((PB, 4), lambda i: (i, 0)),
        ] + [bcast(a) for a in wlist],
        out_specs=pl.BlockSpec((PB, 128), lambda i: (i, 0)),
        out_shape=jax.ShapeDtypeStruct((p, 128), jnp.float32),
    )(features, nv2, coors, *wlist)
    return out


# speedup vs baseline: 14.7311x; 1.2609x over previous
import jax
import jax.numpy as jnp
from jax.experimental import pallas as pl

V = 32
NPOINT = 10
VX = 0.2
VY = 0.2
X_OFFSET = VX / 2 + 0.0
Y_OFFSET = VY / 2 + (-40.0)
RADIUS_LIST = [0.32, 0.36, 0.4]
NSAMPLE_LIST = [6, 8, 10]
BN_EPS = 1e-5

PB = 40


def _fold(p):
    s = p["gamma"] / jnp.sqrt(1.0 + BN_EPS)
    w = (p["w"] * s[:, None]).T
    b = p["b"] * s + p["beta"]
    return w, b[None, :]


def _sa0_kernel_body(feat_ref, nv_ref, coors_ref,
                     w1_ref, w1c_ref, b1_ref, w2_ref, b2_ref, w3_ref, b3_ref,
                     w4a_ref, b4a_ref, w4b_ref, b4b_ref, w4c_ref, b4c_ref,
                     s1w_ref, s1b_ref, s2w_ref, s2b_ref, s3w_ref, s3b_ref,
                     out_ref):
    f32 = jnp.float32
    f = feat_ref[...]
    nv = nv_ref[...].astype(f32)
    coors = coors_ref[...].astype(f32)

    pmean = jnp.sum(f[:, :, :3], axis=1, keepdims=True) / nv[:, :, None]
    fcl = f[:, :, :3] - pmean
    fc0 = f[:, :, 0] - (coors[:, 3][:, None] * VX + X_OFFSET)
    fc1 = f[:, :, 1] - (coors[:, 2][:, None] * VY + Y_OFFSET)
    feats9 = jnp.concatenate(
        [f, fcl, fc0[:, :, None], fc1[:, :, None]], axis=-1)
    jidx = jax.lax.broadcasted_iota(jnp.int32, (PB, V), 1)
    vmask = (nv > jidx.astype(f32)).astype(f32)
    feats9 = feats9 * vmask[:, :, None]

    xyz = feats9[:, :, 0:3]
    data6 = feats9[:, :, 3:9]

    distance = jnp.full((PB, V), 1e10, dtype=f32)
    farth = jnp.zeros((PB, 1), dtype=jnp.int32)
    cents = []
    for _ in range(NPOINT):
        oh = (jidx == farth).astype(f32)
        centroid = jnp.sum(xyz * oh[:, :, None], axis=1)
        cents.append(centroid)
        dx = xyz[:, :, 0] - centroid[:, 0][:, None]
        dy = xyz[:, :, 1] - centroid[:, 1][:, None]
        dz = xyz[:, :, 2] - centroid[:, 2][:, None]
        d = (dx * dx + dy * dy) + dz * dz
        distance = jnp.minimum(distance, d)
        m = jnp.max(distance, axis=1, keepdims=True)
        farth = jnp.min(jnp.where(distance == m, jidx, V), axis=1,
                        keepdims=True)
    new_xyz = jnp.stack(cents, axis=1)

    dds = []
    for c in range(3):
        nxc = new_xyz[:, :, c]
        xc = xyz[:, :, c]
        dds.append(nxc[:, :, None] - xc[:, None, :])
    s4 = (dds[0] * dds[0] + dds[1] * dds[1]) + dds[2] * dds[2]
    s2d = s4.reshape(PB * NPOINT, V)

    jrow = jax.lax.broadcasted_iota(jnp.int32, (V, V), 0)
    jcol = jax.lax.broadcasted_iota(jnp.int32, (V, V), 1)
    lt = (jrow < jcol).astype(f32)

    sels = []
    for r, ns in zip(RADIUS_LIST, NSAMPLE_LIST):
        inr = (s2d <= r * r).astype(f32)
        rank = jnp.dot(inr, lt, preferred_element_type=f32)
        sels.append(inr * (rank < float(ns)).astype(f32))

    rows = PB * NPOINT * V
    pp = jnp.dot(feats9.reshape(PB * V, 9), w1_ref[...],
                 preferred_element_type=f32)
    pc = jnp.dot(new_xyz.reshape(PB * NPOINT, 3), w1c_ref[...],
                 preferred_element_type=f32)
    h = jnp.maximum(pp.reshape(PB, 1, V, 80)
                    - pc.reshape(PB, NPOINT, 1, 80)
                    + b1_ref[...], 0.0).reshape(rows, 80)

    h = jnp.maximum(jnp.dot(h, w2_ref[...], preferred_element_type=f32)
                    + b2_ref[...], 0.0)
    h = jnp.maximum(jnp.dot(h, w3_ref[...], preferred_element_type=f32)
                    + b3_ref[...], 0.0)

    pooled = []
    for (lo, hi), w4, b4, sel in zip(((0, 48), (48, 96), (96, 192)),
                                     (w4a_ref, w4b_ref, w4c_ref),
                                     (b4a_ref, b4b_ref, b4c_ref), sels):
        y = jnp.maximum(jnp.dot(h[:, lo:hi], w4[...],
                                preferred_element_type=f32) + b4[...], 0.0)
        ym = y.reshape(PB * NPOINT, V, y.shape[-1]) * sel[:, :, None]
        pooled.append(jnp.max(ym, axis=1))
    sa0_out = jnp.concatenate(pooled, axis=-1)

    xin = jnp.concatenate([new_xyz.reshape(PB * NPOINT, 3), sa0_out], axis=-1)
    g = jnp.maximum(jnp.dot(xin, s1w_ref[...], preferred_element_type=f32)
                    + s1b_ref[...], 0.0)
    g = jnp.maximum(jnp.dot(g, s2w_ref[...], preferred_element_type=f32)
                    + s2b_ref[...], 0.0)
    g = jnp.maximum(jnp.dot(g, s3w_ref[...], preferred_element_type=f32)
                    + s3b_ref[...], 0.0)
    out_ref[...] = jnp.max(g.reshape(PB, NPOINT, 128), axis=1)


def kernel(features, num_voxels, coors, params):
    p = features.shape[0]
    assert p % PB == 0
    grid = p // PB

    sa0 = [[_fold(layer) for layer in branch] for branch in params["sa0"]]
    sa1 = [_fold(layer) for layer in params["sa1"]]

    w1 = jnp.concatenate([sa0[0][0][0], sa0[1][0][0], sa0[2][0][0]], axis=1)
    b1 = jnp.concatenate([sa0[0][0][1], sa0[1][0][1], sa0[2][0][1]], axis=1)
    w1c = w1[6:9]
    w1 = jnp.concatenate([w1[6:9], w1[0:6]], axis=0)

    def blockdiag(mats):
        rows = sum(m.shape[0] for m in mats)
        cols = sum(m.shape[1] for m in mats)
        out = jnp.zeros((rows, cols), jnp.float32)
        r = c = 0
        for m in mats:
            out = out.at[r:r + m.shape[0], c:c + m.shape[1]].set(m)
            r += m.shape[0]
            c += m.shape[1]
        return out

    w2 = blockdiag([sa0[0][1][0], sa0[1][1][0], sa0[2][1][0]])
    b2 = jnp.concatenate([sa0[0][1][1], sa0[1][1][1], sa0[2][1][1]], axis=1)
    w3 = blockdiag([sa0[0][2][0], sa0[1][2][0], sa0[2][2][0]])
    b3 = jnp.concatenate([sa0[0][2][1], sa0[1][2][1], sa0[2][2][1]], axis=1)
    w4a, b4a = sa0[0][3]
    w4b, b4b = sa0[1][3]
    w4c, b4c = sa0[2][3]
    (s1w, s1b), (s2w, s2b), (s3w, s3b) = sa1

    nv2 = num_voxels.reshape(p, 1)

    def bcast(a):
        return pl.BlockSpec(a.shape, lambda i: (0,) * a.ndim)

    wlist = [w1, w1c, b1, w2, b2, w3, b3, w4a, b4a, w4b, b4b, w4c, b4c,
             s1w, s1b, s2w, s2b, s3w, s3b]

    out = pl.pallas_call(
        _sa0_kernel_body,
        grid=(grid,),
        in_specs=[
            pl.BlockSpec((PB, V, 4), lambda i: (i, 0, 0)),
            pl.BlockSpec((PB, 1), lambda i: (i, 0)),
            pl.BlockSpec((PB, 4), lambda i: (i, 0)),
        ] + [bcast(a) for a in wlist],
        out_specs=pl.BlockSpec((PB, 128), lambda i: (i, 0)),
        out_shape=jax.ShapeDtypeStruct((p, 128), jnp.float32),
    )(features, nv2, coors, *wlist)
    return out


# speedup vs baseline: 14.7356x; 1.0003x over previous
import jax
import jax.numpy as jnp
from jax.experimental import pallas as pl
from jax.experimental.pallas import tpu as pltpu

V = 32
NPOINT = 10
VX = 0.2
VY = 0.2
X_OFFSET = VX / 2 + 0.0
Y_OFFSET = VY / 2 + (-40.0)
RADIUS_LIST = [0.32, 0.36, 0.4]
NSAMPLE_LIST = [6, 8, 10]
BN_EPS = 1e-5

PB = 40


def _fold(p):
    s = p["gamma"] / jnp.sqrt(1.0 + BN_EPS)
    w = (p["w"] * s[:, None]).T
    b = p["b"] * s + p["beta"]
    return w, b[None, :]


def _sa0_kernel_body(feat_ref, nv_ref, coors_ref,
                     w1_ref, w1c_ref, b1_ref, w2_ref, b2_ref, w3_ref, b3_ref,
                     w4a_ref, b4a_ref, w4b_ref, b4b_ref, w4c_ref, b4c_ref,
                     s1w_ref, s1b_ref, s2w_ref, s2b_ref, s3w_ref, s3b_ref,
                     out_ref):
    f32 = jnp.float32
    f = feat_ref[...]
    nv = nv_ref[...].astype(f32)
    coors = coors_ref[...].astype(f32)

    pmean = jnp.sum(f[:, :, :3], axis=1, keepdims=True) / nv[:, :, None]
    fcl = f[:, :, :3] - pmean
    fc0 = f[:, :, 0] - (coors[:, 3][:, None] * VX + X_OFFSET)
    fc1 = f[:, :, 1] - (coors[:, 2][:, None] * VY + Y_OFFSET)
    feats9 = jnp.concatenate(
        [f, fcl, fc0[:, :, None], fc1[:, :, None]], axis=-1)
    jidx = jax.lax.broadcasted_iota(jnp.int32, (PB, V), 1)
    vmask = (nv > jidx.astype(f32)).astype(f32)
    feats9 = feats9 * vmask[:, :, None]

    xyz = feats9[:, :, 0:3]
    data6 = feats9[:, :, 3:9]

    distance = jnp.full((PB, V), 1e10, dtype=f32)
    farth = jnp.zeros((PB, 1), dtype=jnp.int32)
    cents = []
    for _ in range(NPOINT):
        oh = (jidx == farth).astype(f32)
        centroid = jnp.sum(xyz * oh[:, :, None], axis=1)
        cents.append(centroid)
        dx = xyz[:, :, 0] - centroid[:, 0][:, None]
        dy = xyz[:, :, 1] - centroid[:, 1][:, None]
        dz = xyz[:, :, 2] - centroid[:, 2][:, None]
        d = (dx * dx + dy * dy) + dz * dz
        distance = jnp.minimum(distance, d)
        m = jnp.max(distance, axis=1, keepdims=True)
        farth = jnp.min(jnp.where(distance == m, jidx, V), axis=1,
                        keepdims=True)
    new_xyz = jnp.stack(cents, axis=1)

    dds = []
    for c in range(3):
        nxc = new_xyz[:, :, c]
        xc = xyz[:, :, c]
        dds.append(nxc[:, :, None] - xc[:, None, :])
    s4 = (dds[0] * dds[0] + dds[1] * dds[1]) + dds[2] * dds[2]
    s2d = s4.reshape(PB * NPOINT, V)

    jrow = jax.lax.broadcasted_iota(jnp.int32, (V, V), 0)
    jcol = jax.lax.broadcasted_iota(jnp.int32, (V, V), 1)
    lt = (jrow < jcol).astype(f32)

    sels = []
    for r, ns in zip(RADIUS_LIST, NSAMPLE_LIST):
        inr = (s2d <= r * r).astype(f32)
        rank = jnp.dot(inr, lt, preferred_element_type=f32)
        sels.append(inr * (rank < float(ns)).astype(f32))

    rows = PB * NPOINT * V
    pp = jnp.dot(feats9.reshape(PB * V, 9), w1_ref[...],
                 preferred_element_type=f32)
    pc = jnp.dot(new_xyz.reshape(PB * NPOINT, 3), w1c_ref[...],
                 preferred_element_type=f32)
    h = jnp.maximum(pp.reshape(PB, 1, V, 80)
                    - pc.reshape(PB, NPOINT, 1, 80)
                    + b1_ref[...], 0.0).reshape(rows, 80)

    h = jnp.maximum(jnp.dot(h, w2_ref[...], preferred_element_type=f32)
                    + b2_ref[...], 0.0)
    h = jnp.maximum(jnp.dot(h, w3_ref[...], preferred_element_type=f32)
                    + b3_ref[...], 0.0)

    pooled = []
    for (lo, hi), w4, b4, sel in zip(((0, 48), (48, 96), (96, 192)),
                                     (w4a_ref, w4b_ref, w4c_ref),
                                     (b4a_ref, b4b_ref, b4c_ref), sels):
        y = jnp.maximum(jnp.dot(h[:, lo:hi], w4[...],
                                preferred_element_type=f32) + b4[...], 0.0)
        ym = y.reshape(PB * NPOINT, V, y.shape[-1]) * sel[:, :, None]
        pooled.append(jnp.max(ym, axis=1))
    sa0_out = jnp.concatenate(pooled, axis=-1)

    xin = jnp.concatenate([new_xyz.reshape(PB * NPOINT, 3), sa0_out], axis=-1)
    g = jnp.maximum(jnp.dot(xin, s1w_ref[...], preferred_element_type=f32)
                    + s1b_ref[...], 0.0)
    g = jnp.maximum(jnp.dot(g, s2w_ref[...], preferred_element_type=f32)
                    + s2b_ref[...], 0.0)
    g = jnp.maximum(jnp.dot(g, s3w_ref[...], preferred_element_type=f32)
                    + s3b_ref[...], 0.0)
    out_ref[...] = jnp.max(g.reshape(PB, NPOINT, 128), axis=1)


def kernel(features, num_voxels, coors, params):
    p = features.shape[0]
    assert p % PB == 0
    grid = p // PB

    sa0 = [[_fold(layer) for layer in branch] for branch in params["sa0"]]
    sa1 = [_fold(layer) for layer in params["sa1"]]

    w1 = jnp.concatenate([sa0[0][0][0], sa0[1][0][0], sa0[2][0][0]], axis=1)
    b1 = jnp.concatenate([sa0[0][0][1], sa0[1][0][1], sa0[2][0][1]], axis=1)
    w1c = w1[6:9]
    w1 = jnp.concatenate([w1[6:9], w1[0:6]], axis=0)

    def blockdiag(mats):
        rows = sum(m.shape[0] for m in mats)
        cols = sum(m.shape[1] for m in mats)
        out = jnp.zeros((rows, cols), jnp.float32)
        r = c = 0
        for m in mats:
            out = out.at[r:r + m.shape[0], c:c + m.shape[1]].set(m)
            r += m.shape[0]
            c += m.shape[1]
        return out

    w2 = blockdiag([sa0[0][1][0], sa0[1][1][0], sa0[2][1][0]])
    b2 = jnp.concatenate([sa0[0][1][1], sa0[1][1][1], sa0[2][1][1]], axis=1)
    w3 = blockdiag([sa0[0][2][0], sa0[1][2][0], sa0[2][2][0]])
    b3 = jnp.concatenate([sa0[0][2][1], sa0[1][2][1], sa0[2][2][1]], axis=1)
    w4a, b4a = sa0[0][3]
    w4b, b4b = sa0[1][3]
    w4c, b4c = sa0[2][3]
    (s1w, s1b), (s2w, s2b), (s3w, s3b) = sa1

    nv2 = num_voxels.reshape(p, 1)

    def bcast(a):
        return pl.BlockSpec(a.shape, lambda i: (0,) * a.ndim)

    wlist = [w1, w1c, b1, w2, b2, w3, b3, w4a, b4a, w4b, b4b, w4c, b4c,
             s1w, s1b, s2w, s2b, s3w, s3b]

    out = pl.pallas_call(
        _sa0_kernel_body,
        grid=(grid,),
        in_specs=[
            pl.BlockSpec((PB, V, 4), lambda i: (i, 0, 0)),
            pl.BlockSpec((PB, 1), lambda i: (i, 0)),
            pl.BlockSpec((PB, 4), lambda i: (i, 0)),
        ] + [bcast(a) for a in wlist],
        out_specs=pl.BlockSpec((PB, 128), lambda i: (i, 0)),
        out_shape=jax.ShapeDtypeStruct((p, 128), jnp.float32),
        compiler_params=pltpu.CompilerParams(
            dimension_semantics=("parallel",)),
    )(features, nv2, coors, *wlist)
    return out


# speedup vs baseline: 15.5368x; 1.0544x over previous
import jax
import jax.numpy as jnp
from jax.experimental import pallas as pl
from jax.experimental.pallas import tpu as pltpu

V = 32
NPOINT = 10
VX = 0.2
VY = 0.2
X_OFFSET = VX / 2 + 0.0
Y_OFFSET = VY / 2 + (-40.0)
RADIUS_LIST = [0.32, 0.36, 0.4]
NSAMPLE_LIST = [6, 8, 10]
BN_EPS = 1e-5

PB = 40


def _fold(p):
    s = p["gamma"] / jnp.sqrt(1.0 + BN_EPS)
    w = (p["w"] * s[:, None]).T
    b = p["b"] * s + p["beta"]
    return w, b[None, :]


def _sa0_kernel_body(feat_ref, nv_ref, coors_ref,
                     w1_ref, w1c_ref, b1_ref, w2_ref, b2_ref, w3_ref, b3_ref,
                     w4a_ref, b4a_ref, w4b_ref, b4b_ref, w4c_ref, b4c_ref,
                     s1w_ref, s1b_ref, s2w_ref, s2b_ref, s3w_ref, s3b_ref,
                     out_ref):
    f32 = jnp.float32
    f = feat_ref[...]
    nv = nv_ref[...].astype(f32)
    coors = coors_ref[...].astype(f32)

    pmean = jnp.sum(f[:, :, :3], axis=1, keepdims=True) / nv[:, :, None]
    fcl = f[:, :, :3] - pmean
    fc0 = f[:, :, 0] - (coors[:, 3][:, None] * VX + X_OFFSET)
    fc1 = f[:, :, 1] - (coors[:, 2][:, None] * VY + Y_OFFSET)
    feats9 = jnp.concatenate(
        [f, fcl, fc0[:, :, None], fc1[:, :, None]], axis=-1)
    jidx = jax.lax.broadcasted_iota(jnp.int32, (PB, V), 1)
    vmask = (nv > jidx.astype(f32)).astype(f32)
    feats9 = feats9 * vmask[:, :, None]

    xyz = feats9[:, :, 0:3]
    data6 = feats9[:, :, 3:9]

    distance = jnp.full((PB, V), 1e10, dtype=f32)
    farth = jnp.zeros((PB, 1), dtype=jnp.int32)
    cents = []
    for _ in range(NPOINT):
        oh = (jidx == farth).astype(f32)
        centroid = jnp.sum(xyz * oh[:, :, None], axis=1)
        cents.append(centroid)
        dx = xyz[:, :, 0] - centroid[:, 0][:, None]
        dy = xyz[:, :, 1] - centroid[:, 1][:, None]
        dz = xyz[:, :, 2] - centroid[:, 2][:, None]
        d = (dx * dx + dy * dy) + dz * dz
        distance = jnp.minimum(distance, d)
        m = jnp.max(distance, axis=1, keepdims=True)
        farth = jnp.min(jnp.where(distance == m, jidx, V), axis=1,
                        keepdims=True)
    new_xyz = jnp.stack(cents, axis=1)

    dds = []
    for c in range(3):
        nxc = new_xyz[:, :, c]
        xc = xyz[:, :, c]
        dds.append(nxc[:, :, None] - xc[:, None, :])
    s4 = (dds[0] * dds[0] + dds[1] * dds[1]) + dds[2] * dds[2]
    s2d = s4.reshape(PB * NPOINT, V)

    jrow = jax.lax.broadcasted_iota(jnp.int32, (V, V), 0)
    jcol = jax.lax.broadcasted_iota(jnp.int32, (V, V), 1)
    lt = (jrow < jcol).astype(f32)

    sels = []
    for r, ns in zip(RADIUS_LIST, NSAMPLE_LIST):
        inr = (s2d <= r * r).astype(f32)
        rank = jnp.dot(inr, lt, preferred_element_type=f32)
        sels.append(inr * (rank < float(ns)).astype(f32))

    rows = PB * NPOINT * V
    pp = jnp.dot(feats9.reshape(PB * V, 9), w1_ref[...],
                 preferred_element_type=f32)
    pc = jnp.dot(new_xyz.reshape(PB * NPOINT, 3), w1c_ref[...],
                 preferred_element_type=f32)
    bf16 = jnp.bfloat16
    h = jnp.maximum(pp.reshape(PB, 1, V, 80)
                    - pc.reshape(PB, NPOINT, 1, 80)
                    + b1_ref[...], 0.0).reshape(rows, 80).astype(bf16)

    h = (jnp.maximum(jnp.dot(h, w2_ref[...], preferred_element_type=f32)
                     + b2_ref[...], 0.0)).astype(bf16)
    h = (jnp.maximum(jnp.dot(h, w3_ref[...], preferred_element_type=f32)
                     + b3_ref[...], 0.0)).astype(bf16)

    pooled = []
    for (lo, hi), w4, b4, sel in zip(((0, 48), (48, 96), (96, 192)),
                                     (w4a_ref, w4b_ref, w4c_ref),
                                     (b4a_ref, b4b_ref, b4c_ref), sels):
        y = (jnp.maximum(jnp.dot(h[:, lo:hi], w4[...],
                                 preferred_element_type=f32)
                         + b4[...], 0.0)).astype(bf16)
        ym = y.reshape(PB * NPOINT, V, y.shape[-1]) * sel[:, :, None].astype(bf16)
        pooled.append(jnp.max(ym, axis=1))
    sa0_out = jnp.concatenate(pooled, axis=-1).astype(f32)

    xin = jnp.concatenate([new_xyz.reshape(PB * NPOINT, 3), sa0_out], axis=-1)
    g = jnp.maximum(jnp.dot(xin, s1w_ref[...], preferred_element_type=f32)
                    + s1b_ref[...], 0.0)
    g = jnp.maximum(jnp.dot(g, s2w_ref[...], preferred_element_type=f32)
                    + s2b_ref[...], 0.0)
    g = jnp.maximum(jnp.dot(g, s3w_ref[...], preferred_element_type=f32)
                    + s3b_ref[...], 0.0)
    out_ref[...] = jnp.max(g.reshape(PB, NPOINT, 128), axis=1)


def kernel(features, num_voxels, coors, params):
    p = features.shape[0]
    assert p % PB == 0
    grid = p // PB

    sa0 = [[_fold(layer) for layer in branch] for branch in params["sa0"]]
    sa1 = [_fold(layer) for layer in params["sa1"]]

    w1 = jnp.concatenate([sa0[0][0][0], sa0[1][0][0], sa0[2][0][0]], axis=1)
    b1 = jnp.concatenate([sa0[0][0][1], sa0[1][0][1], sa0[2][0][1]], axis=1)
    w1c = w1[6:9]
    w1 = jnp.concatenate([w1[6:9], w1[0:6]], axis=0)

    def blockdiag(mats):
        rows = sum(m.shape[0] for m in mats)
        cols = sum(m.shape[1] for m in mats)
        out = jnp.zeros((rows, cols), jnp.float32)
        r = c = 0
        for m in mats:
            out = out.at[r:r + m.shape[0], c:c + m.shape[1]].set(m)
            r += m.shape[0]
            c += m.shape[1]
        return out

    w2 = blockdiag([sa0[0][1][0], sa0[1][1][0], sa0[2][1][0]])
    b2 = jnp.concatenate([sa0[0][1][1], sa0[1][1][1], sa0[2][1][1]], axis=1)
    w3 = blockdiag([sa0[0][2][0], sa0[1][2][0], sa0[2][2][0]])
    b3 = jnp.concatenate([sa0[0][2][1], sa0[1][2][1], sa0[2][2][1]], axis=1)
    w4a, b4a = sa0[0][3]
    w4b, b4b = sa0[1][3]
    w4c, b4c = sa0[2][3]
    (s1w, s1b), (s2w, s2b), (s3w, s3b) = sa1
    bf = jnp.bfloat16
    w2, w3 = w2.astype(bf), w3.astype(bf)
    w4a, w4b, w4c = w4a.astype(bf), w4b.astype(bf), w4c.astype(bf)

    nv2 = num_voxels.reshape(p, 1)

    def bcast(a):
        return pl.BlockSpec(a.shape, lambda i: (0,) * a.ndim)

    wlist = [w1, w1c, b1, w2, b2, w3, b3, w4a, b4a, w4b, b4b, w4c, b4c,
             s1w, s1b, s2w, s2b, s3w, s3b]

    out = pl.pallas_call(
        _sa0_kernel_body,
        grid=(grid,),
        in_specs=[
            pl.BlockSpec((PB, V, 4), lambda i: (i, 0, 0)),
            pl.BlockSpec((PB, 1), lambda i: (i, 0)),
            pl.BlockSpec((PB, 4), lambda i: (i, 0)),
        ] + [bcast(a) for a in wlist],
        out_specs=pl.BlockSpec((PB, 128), lambda i: (i, 0)),
        out_shape=jax.ShapeDtypeStruct((p, 128), jnp.float32),
        compiler_params=pltpu.CompilerParams(
            dimension_semantics=("parallel",)),
    )(features, nv2, coors, *wlist)
    return out
